# baseline (speedup 1.0000x reference)
import functools

import jax
import jax.numpy as jnp
from jax import lax
from jax.experimental import pallas as pl
from jax.experimental.pallas import tpu as pltpu
from jax.experimental.pallas import tpu_sc as plsc

D = 128
BLK = 128
NW = 32
NSUB = 16


def _prep_body(beta_ref, h_ref, w_ref, z_ref, zh_ref):
    z = lax.dot_general(h_ref[...], w_ref[...], (((1,), (1,)), ((), ())),
                        preferred_element_type=jnp.float32)
    z_ref[...] = z
    nrm = jnp.sqrt(jnp.sum(z * z, axis=1, keepdims=True))
    scale = jnp.sqrt(beta_ref[0, 0]) / jnp.maximum(nrm, 1e-6)
    zh_ref[...] = (z * scale).astype(jnp.bfloat16)


def _prep(h, W, beta):
    n = h.shape[0]
    rb = 1000
    grid = n // rb
    return pl.pallas_call(
        _prep_body,
        grid=(grid,),
        in_specs=[
            pl.BlockSpec((1, 1), lambda i: (0, 0)),
            pl.BlockSpec((rb, D), lambda i: (i, 0)),
            pl.BlockSpec((D, D), lambda i: (0, 0)),
        ],
        out_specs=[
            pl.BlockSpec((rb, D), lambda i: (i, 0)),
            pl.BlockSpec((rb, D), lambda i: (i, 0)),
        ],
        out_shape=[
            jax.ShapeDtypeStruct((n, D), jnp.float32),
            jax.ShapeDtypeStruct((n, D), jnp.bfloat16),
        ],
    )(beta.reshape(1, 1), h, W)


def _gram_body(a_ref, b_ref, o_ref):
    o_ref[...] = lax.dot_general(a_ref[...], b_ref[...],
                                 (((1,), (1,)), ((), ())),
                                 preferred_element_type=jnp.float32)


def _gram_body3(a_ref, b_ref, o_ref):
    o_ref[0, ...] = lax.dot_general(a_ref[...], b_ref[...],
                                    (((1,), (1,)), ((), ())),
                                    preferred_element_type=jnp.float32)


def _gram(zh):
    n = zh.shape[0]
    bm = 1000
    ncb = pl.cdiv(n, D)
    return pl.pallas_call(
        _gram_body3,
        grid=(n // bm, ncb),
        in_specs=[
            pl.BlockSpec((bm, D), lambda i, j: (i, 0)),
            pl.BlockSpec((D, D), lambda i, j: (j, 0)),
        ],
        out_specs=pl.BlockSpec((1, bm, D), lambda i, j: (j, i, 0)),
        out_shape=jax.ShapeDtypeStruct((ncb, n, D), jnp.float32),
    )(zh, zh)


def _make_k1(e, n, npad):
    nblk = e // BLK
    iters = pl.cdiv(nblk, NW)
    sl = npad // NSUB
    mesh = plsc.VectorSubcoreMesh(core_axis_name="c", subcore_axis_name="s", num_cores=2, num_subcores=16)

    @functools.partial(
        pl.kernel,
        mesh=mesh,
        compiler_params=pltpu.CompilerParams(needs_layout_passes=False),
        out_type=(
            jax.ShapeDtypeStruct((e,), jnp.float32),
            jax.ShapeDtypeStruct((npad,), jnp.float32),
            jax.ShapeDtypeStruct((npad,), jnp.float32),
        ),
        scratch_types=[
            pltpu.VMEM((BLK,), jnp.int32),
            pltpu.VMEM((BLK,), jnp.int32),
            pltpu.VMEM((BLK,), jnp.int32),
            pltpu.VMEM((BLK,), jnp.int32),
            pltpu.VMEM((BLK, D), jnp.float32),
            pltpu.VMEM((BLK,), jnp.float32),
            pltpu.VMEM((sl,), jnp.float32),
            pltpu.VMEM_SHARED((npad,), jnp.float32),
            pltpu.SemaphoreType.DMA,
        ],
    )
    def k1(g2_hbm, src_hbm, dst_hbm, w_hbm, s0_hbm, s1_hbm,
           src_v, dst_v, row_v, cmod_v, gst_v, w_v, sbuf_v, s_sh, sem):
        cid = lax.axis_index("c")
        sid = lax.axis_index("s")
        wid = cid * NSUB + sid

        zero16 = jnp.zeros((16,), jnp.float32)
        def zbody(i, c):
            sbuf_v[pl.ds(i * 16, 16)] = zero16
            return c
        lax.fori_loop(0, sl // 16, zbody, 0)
        pltpu.sync_copy(sbuf_v, s_sh.at[pl.ds(sid * sl, sl)])
        plsc.subcore_barrier()

        nconst = jnp.full((16,), n, jnp.int32)

        def block_body(i, c):
            b = wid + i * NW

            @pl.when(b < nblk)
            def _():
                base = b * BLK
                pltpu.sync_copy(src_hbm.at[pl.ds(base, BLK)], src_v)
                pltpu.sync_copy(dst_hbm.at[pl.ds(base, BLK)], dst_v)
                seven = jnp.full((16,), 7, jnp.int32)
                mask7 = jnp.full((16,), D - 1, jnp.int32)
                for g in range(BLK // 16):
                    ix = pl.ds(g * 16, 16)
                    dv = dst_v[ix]
                    row_v[ix] = lax.shift_right_logical(dv, seven) * nconst \
                        + src_v[ix]
                    cmod_v[ix] = lax.bitwise_and(dv, mask7)
                pltpu.async_copy(g2_hbm.at[row_v], gst_v, sem).wait()
                for g in range(BLK // 16):
                    ix = pl.ds(g * 16, 16)
                    rows = lax.iota(jnp.int32, 16) + g * 16
                    val = plsc.load_gather(gst_v, [rows, cmod_v[ix]])
                    w_v[ix] = jnp.exp(val)
                pltpu.sync_copy(w_v, w_hbm.at[pl.ds(base, BLK)])
                pltpu.sync_copy(w_v, s_sh.at[dst_v], add=True)
            return c

        lax.fori_loop(0, iters, block_body, 0)
        plsc.subcore_barrier()

        pltpu.sync_copy(s_sh.at[pl.ds(sid * sl, sl)], sbuf_v)

        @pl.when(cid == 0)
        def _():
            pltpu.sync_copy(sbuf_v, s0_hbm.at[pl.ds(sid * sl, sl)])

        @pl.when(cid == 1)
        def _():
            pltpu.sync_copy(sbuf_v, s1_hbm.at[pl.ds(sid * sl, sl)])

    return k1


def _make_k2(e, npad):
    nblk = e // BLK
    iters = pl.cdiv(nblk, NW)
    sl = npad // NSUB
    rows_per_copy = 128
    mesh = plsc.VectorSubcoreMesh(core_axis_name="c", subcore_axis_name="s", num_cores=2, num_subcores=16)

    @functools.partial(
        pl.kernel,
        mesh=mesh,
        compiler_params=pltpu.CompilerParams(needs_layout_passes=False),
        out_type=(
            jax.ShapeDtypeStruct((npad, D), jnp.float32),
            jax.ShapeDtypeStruct((npad, D), jnp.float32),
        ),
        scratch_types=[
            pltpu.VMEM((BLK,), jnp.int32),
            pltpu.VMEM((BLK,), jnp.int32),
            pltpu.VMEM((BLK,), jnp.float32),
            pltpu.VMEM((BLK,), jnp.float32),
            pltpu.VMEM((BLK, D), jnp.float32),
            pltpu.VMEM((npad,), jnp.float32),
            pltpu.VMEM((npad,), jnp.float32),
            pltpu.VMEM_SHARED((npad, D), jnp.float32),
            pltpu.SemaphoreType.DMA,
        ],
    )
    def k2(z_hbm, src_hbm, dst_hbm, w_hbm, s0_hbm, s1_hbm,
           out0_hbm, out1_hbm,
           src_v, dst_v, w_v, al_v, zr_v, s_v, tmp_v, out_sh, sem):
        cid = lax.axis_index("c")
        sid = lax.axis_index("s")
        wid = cid * NSUB + sid

        pltpu.sync_copy(s0_hbm, s_v)
        pltpu.sync_copy(s1_hbm, tmp_v)

        def sbody(i, c):
            ix = pl.ds(i * 16, 16)
            s_v[ix] = s_v[ix] + tmp_v[ix]
            return c
        lax.fori_loop(0, npad // 16, sbody, 0)

        zero16 = jnp.zeros((16,), jnp.float32)
        def zbody(i, c):
            r = i // 8
            cchunk = i % 8
            zr_v[r, pl.ds(cchunk * 16, 16)] = zero16
            return c
        lax.fori_loop(0, rows_per_copy * 8, zbody, 0)
        for j in range(sl // rows_per_copy):
            pltpu.sync_copy(
                zr_v, out_sh.at[pl.ds(sid * sl + j * rows_per_copy,
                                      rows_per_copy)])
        plsc.subcore_barrier()

        def block_body(i, c):
            b = wid + i * NW

            @pl.when(b < nblk)
            def _():
                base = b * BLK
                pltpu.sync_copy(src_hbm.at[pl.ds(base, BLK)], src_v)
                pltpu.sync_copy(dst_hbm.at[pl.ds(base, BLK)], dst_v)
                pltpu.sync_copy(w_hbm.at[pl.ds(base, BLK)], w_v)
                cp = pltpu.async_copy(z_hbm.at[src_v], zr_v, sem)
                for g in range(BLK // 16):
                    ix = pl.ds(g * 16, 16)
                    dv = dst_v[ix]
                    sg = plsc.load_gather(s_v, [dv])
                    al_v[ix] = w_v[ix] / sg
                cp.wait()

                def ebody(ei, c2):
                    ab = plsc.load_gather(
                        al_v, [jnp.full((16,), ei, jnp.int32)])
                    for cchunk in range(8):
                        ix = pl.ds(cchunk * 16, 16)
                        zr_v[ei, ix] = zr_v[ei, ix] * ab
                    return c2

                lax.fori_loop(0, BLK, ebody, 0)
                pltpu.sync_copy(zr_v, out_sh.at[dst_v], add=True)
            return c

        lax.fori_loop(0, iters, block_body, 0)
        plsc.subcore_barrier()

        for j in range(sl // rows_per_copy):
            r0 = sid * sl + j * rows_per_copy
            pltpu.sync_copy(out_sh.at[pl.ds(r0, rows_per_copy)], zr_v)

            @pl.when(cid == 0)
            def _():
                pltpu.sync_copy(zr_v, out0_hbm.at[pl.ds(r0, rows_per_copy)])

            @pl.when(cid == 1)
            def _():
                pltpu.sync_copy(zr_v, out1_hbm.at[pl.ds(r0, rows_per_copy)])

    return k2


def _add_body(a_ref, b_ref, o_ref):
    o_ref[...] = a_ref[...] + b_ref[...]


def _final_add(a, b, n):
    rb = 80
    grid = n // rb
    return pl.pallas_call(
        _add_body,
        grid=(grid,),
        in_specs=[
            pl.BlockSpec((rb, D), lambda i: (i, 0)),
            pl.BlockSpec((rb, D), lambda i: (i, 0)),
        ],
        out_specs=pl.BlockSpec((rb, D), lambda i: (i, 0)),
        out_shape=jax.ShapeDtypeStruct((n, D), jnp.float32),
    )(a, b)


def kernel(h, edge_index, W, beta):
    n = h.shape[0]
    e = edge_index.shape[1]
    npad = ((n + 2047) // 2048) * 2048

    z, zh = _prep(h, W, beta)
    g2 = _gram(zh)
    g2 = g2.reshape(g2.shape[0] * n, D)
    src = edge_index[0]
    dst = edge_index[1]
    w, s0, s1 = _make_k1(e, n, npad)(g2, src, dst)
    out0, out1 = _make_k2(e, npad)(z, src, dst, w, s0, s1)
    return _final_add(out0, out1, n)


# speedup vs baseline: 1.4670x; 1.4670x over previous
import functools

import jax
import jax.numpy as jnp
from jax import lax
from jax.experimental import pallas as pl
from jax.experimental.pallas import tpu as pltpu
from jax.experimental.pallas import tpu_sc as plsc

D = 128
BLK = 128
NW = 32
NSUB = 16


def _prep_body(beta_ref, h_ref, w_ref, z_ref, zh_ref):
    z = lax.dot_general(h_ref[...], w_ref[...], (((1,), (1,)), ((), ())),
                        preferred_element_type=jnp.float32)
    z_ref[...] = z
    nrm = jnp.sqrt(jnp.sum(z * z, axis=1, keepdims=True))
    scale = jnp.sqrt(beta_ref[0, 0]) / jnp.maximum(nrm, 1e-6)
    zh_ref[...] = (z * scale).astype(jnp.bfloat16)


def _prep(h, W, beta):
    n = h.shape[0]
    rb = 1000
    grid = n // rb
    return pl.pallas_call(
        _prep_body,
        grid=(grid,),
        in_specs=[
            pl.BlockSpec((1, 1), lambda i: (0, 0)),
            pl.BlockSpec((rb, D), lambda i: (i, 0)),
            pl.BlockSpec((D, D), lambda i: (0, 0)),
        ],
        out_specs=[
            pl.BlockSpec((rb, D), lambda i: (i, 0)),
            pl.BlockSpec((rb, D), lambda i: (i, 0)),
        ],
        out_shape=[
            jax.ShapeDtypeStruct((n, D), jnp.float32),
            jax.ShapeDtypeStruct((n, D), jnp.bfloat16),
        ],
    )(beta.reshape(1, 1), h, W)


def _gram_body(a_ref, b_ref, o_ref):
    o_ref[...] = lax.dot_general(a_ref[...], b_ref[...],
                                 (((1,), (1,)), ((), ())),
                                 preferred_element_type=jnp.float32)


JB = 16


def _gram_body3(a_ref, b_ref, o_ref):
    for k in range(JB):
        o_ref[k, ...] = lax.dot_general(
            a_ref[...], b_ref[pl.ds(k * D, D), :],
            (((1,), (1,)), ((), ())),
            preferred_element_type=jnp.float32)


def _gram(zh):
    n = zh.shape[0]
    bm = 1000
    ncb = pl.cdiv(n, JB * D)
    return pl.pallas_call(
        _gram_body3,
        grid=(n // bm, ncb),
        in_specs=[
            pl.BlockSpec((bm, D), lambda i, j: (i, 0)),
            pl.BlockSpec((JB * D, D), lambda i, j: (j, 0)),
        ],
        out_specs=pl.BlockSpec((JB, bm, D), lambda i, j: (j, i, 0)),
        out_shape=jax.ShapeDtypeStruct((ncb * JB, n, D), jnp.float32),
    )(zh, zh)


def _make_k1(e, n, npad):
    nblk = e // BLK
    iters = pl.cdiv(nblk, NW)
    sl = npad // NSUB
    mesh = plsc.VectorSubcoreMesh(core_axis_name="c", subcore_axis_name="s", num_cores=2, num_subcores=16)

    @functools.partial(
        pl.kernel,
        mesh=mesh,
        compiler_params=pltpu.CompilerParams(needs_layout_passes=False),
        out_type=(
            jax.ShapeDtypeStruct((e,), jnp.float32),
            jax.ShapeDtypeStruct((npad,), jnp.float32),
            jax.ShapeDtypeStruct((npad,), jnp.float32),
        ),
        scratch_types=[
            pltpu.VMEM((BLK,), jnp.int32),
            pltpu.VMEM((BLK,), jnp.int32),
            pltpu.VMEM((BLK,), jnp.int32),
            pltpu.VMEM((BLK,), jnp.int32),
            pltpu.VMEM((BLK, D), jnp.float32),
            pltpu.VMEM((BLK,), jnp.float32),
            pltpu.VMEM((sl,), jnp.float32),
            pltpu.VMEM_SHARED((npad,), jnp.float32),
            pltpu.SemaphoreType.DMA,
        ],
    )
    def k1(g2_hbm, src_hbm, dst_hbm, w_hbm, s0_hbm, s1_hbm,
           src_v, dst_v, row_v, cmod_v, gst_v, w_v, sbuf_v, s_sh, sem):
        cid = lax.axis_index("c")
        sid = lax.axis_index("s")
        wid = cid * NSUB + sid

        zero16 = jnp.zeros((16,), jnp.float32)
        def zbody(i, c):
            sbuf_v[pl.ds(i * 16, 16)] = zero16
            return c
        lax.fori_loop(0, sl // 16, zbody, 0)
        pltpu.sync_copy(sbuf_v, s_sh.at[pl.ds(sid * sl, sl)])
        plsc.subcore_barrier()

        nconst = jnp.full((16,), n, jnp.int32)

        def block_body(i, c):
            b = wid + i * NW

            @pl.when(b < nblk)
            def _():
                base = b * BLK
                pltpu.sync_copy(src_hbm.at[pl.ds(base, BLK)], src_v)
                pltpu.sync_copy(dst_hbm.at[pl.ds(base, BLK)], dst_v)
                seven = jnp.full((16,), 7, jnp.int32)
                mask7 = jnp.full((16,), D - 1, jnp.int32)
                for g in range(BLK // 16):
                    ix = pl.ds(g * 16, 16)
                    dv = dst_v[ix]
                    row_v[ix] = lax.shift_right_logical(dv, seven) * nconst \
                        + src_v[ix]
                    cmod_v[ix] = lax.bitwise_and(dv, mask7)
                pltpu.async_copy(g2_hbm.at[row_v], gst_v, sem).wait()
                for g in range(BLK // 16):
                    ix = pl.ds(g * 16, 16)
                    rows = lax.iota(jnp.int32, 16) + g * 16
                    val = plsc.load_gather(gst_v, [rows, cmod_v[ix]])
                    w_v[ix] = jnp.exp(val)
                pltpu.sync_copy(w_v, w_hbm.at[pl.ds(base, BLK)])
                pltpu.sync_copy(w_v, s_sh.at[dst_v], add=True)
            return c

        lax.fori_loop(0, iters, block_body, 0)
        plsc.subcore_barrier()

        pltpu.sync_copy(s_sh.at[pl.ds(sid * sl, sl)], sbuf_v)

        @pl.when(cid == 0)
        def _():
            pltpu.sync_copy(sbuf_v, s0_hbm.at[pl.ds(sid * sl, sl)])

        @pl.when(cid == 1)
        def _():
            pltpu.sync_copy(sbuf_v, s1_hbm.at[pl.ds(sid * sl, sl)])

    return k1


def _make_k2(e, npad):
    nblk = e // BLK
    iters = pl.cdiv(nblk, NW)
    sl = npad // NSUB
    rows_per_copy = 128
    mesh = plsc.VectorSubcoreMesh(core_axis_name="c", subcore_axis_name="s", num_cores=2, num_subcores=16)

    @functools.partial(
        pl.kernel,
        mesh=mesh,
        compiler_params=pltpu.CompilerParams(needs_layout_passes=False),
        out_type=(
            jax.ShapeDtypeStruct((npad, D), jnp.float32),
            jax.ShapeDtypeStruct((npad, D), jnp.float32),
        ),
        scratch_types=[
            pltpu.VMEM((BLK,), jnp.int32),
            pltpu.VMEM((BLK,), jnp.int32),
            pltpu.VMEM((BLK,), jnp.float32),
            pltpu.VMEM((BLK,), jnp.float32),
            pltpu.VMEM((BLK, D), jnp.float32),
            pltpu.VMEM((npad,), jnp.float32),
            pltpu.VMEM((npad,), jnp.float32),
            pltpu.VMEM_SHARED((npad, D), jnp.float32),
            pltpu.SemaphoreType.DMA,
        ],
    )
    def k2(z_hbm, src_hbm, dst_hbm, w_hbm, s0_hbm, s1_hbm,
           out0_hbm, out1_hbm,
           src_v, dst_v, w_v, al_v, zr_v, s_v, tmp_v, out_sh, sem):
        cid = lax.axis_index("c")
        sid = lax.axis_index("s")
        wid = cid * NSUB + sid

        pltpu.sync_copy(s0_hbm, s_v)
        pltpu.sync_copy(s1_hbm, tmp_v)

        def sbody(i, c):
            ix = pl.ds(i * 16, 16)
            s_v[ix] = s_v[ix] + tmp_v[ix]
            return c
        lax.fori_loop(0, npad // 16, sbody, 0)

        zero16 = jnp.zeros((16,), jnp.float32)
        def zbody(i, c):
            r = i // 8
            cchunk = i % 8
            zr_v[r, pl.ds(cchunk * 16, 16)] = zero16
            return c
        lax.fori_loop(0, rows_per_copy * 8, zbody, 0)
        for j in range(sl // rows_per_copy):
            pltpu.sync_copy(
                zr_v, out_sh.at[pl.ds(sid * sl + j * rows_per_copy,
                                      rows_per_copy)])
        plsc.subcore_barrier()

        def block_body(i, c):
            b = wid + i * NW

            @pl.when(b < nblk)
            def _():
                base = b * BLK
                pltpu.sync_copy(src_hbm.at[pl.ds(base, BLK)], src_v)
                pltpu.sync_copy(dst_hbm.at[pl.ds(base, BLK)], dst_v)
                pltpu.sync_copy(w_hbm.at[pl.ds(base, BLK)], w_v)
                cp = pltpu.async_copy(z_hbm.at[src_v], zr_v, sem)
                for g in range(BLK // 16):
                    ix = pl.ds(g * 16, 16)
                    dv = dst_v[ix]
                    sg = plsc.load_gather(s_v, [dv])
                    al_v[ix] = w_v[ix] / sg
                cp.wait()

                def ebody(ei, c2):
                    ab = plsc.load_gather(
                        al_v, [jnp.full((16,), ei, jnp.int32)])
                    for cchunk in range(8):
                        ix = pl.ds(cchunk * 16, 16)
                        zr_v[ei, ix] = zr_v[ei, ix] * ab
                    return c2

                lax.fori_loop(0, BLK, ebody, 0)
                pltpu.sync_copy(zr_v, out_sh.at[dst_v], add=True)
            return c

        lax.fori_loop(0, iters, block_body, 0)
        plsc.subcore_barrier()

        for j in range(sl // rows_per_copy):
            r0 = sid * sl + j * rows_per_copy
            pltpu.sync_copy(out_sh.at[pl.ds(r0, rows_per_copy)], zr_v)

            @pl.when(cid == 0)
            def _():
                pltpu.sync_copy(zr_v, out0_hbm.at[pl.ds(r0, rows_per_copy)])

            @pl.when(cid == 1)
            def _():
                pltpu.sync_copy(zr_v, out1_hbm.at[pl.ds(r0, rows_per_copy)])

    return k2


def _add_body(a_ref, b_ref, o_ref):
    o_ref[...] = a_ref[...] + b_ref[...]


def _final_add(a, b, n):
    rb = 80
    grid = n // rb
    return pl.pallas_call(
        _add_body,
        grid=(grid,),
        in_specs=[
            pl.BlockSpec((rb, D), lambda i: (i, 0)),
            pl.BlockSpec((rb, D), lambda i: (i, 0)),
        ],
        out_specs=pl.BlockSpec((rb, D), lambda i: (i, 0)),
        out_shape=jax.ShapeDtypeStruct((n, D), jnp.float32),
    )(a, b)


def kernel(h, edge_index, W, beta):
    n = h.shape[0]
    e = edge_index.shape[1]
    npad = ((n + 2047) // 2048) * 2048

    z, zh = _prep(h, W, beta)
    g2 = _gram(zh)
    g2 = g2.reshape(g2.shape[0] * n, D)
    src = edge_index[0]
    dst = edge_index[1]
    w, s0, s1 = _make_k1(e, n, npad)(g2, src, dst)
    out0, out1 = _make_k2(e, npad)(z, src, dst, w, s0, s1)
    return _final_add(out0, out1, n)


# speedup vs baseline: 1.9372x; 1.3205x over previous
import functools

import jax
import jax.numpy as jnp
from jax import lax
from jax.experimental import pallas as pl
from jax.experimental.pallas import tpu as pltpu
from jax.experimental.pallas import tpu_sc as plsc

D = 128
BLK = 128
NW = 32
NSUB = 16


def _prep_body(beta_ref, h_ref, w_ref, z_ref, zh_ref):
    z = lax.dot_general(h_ref[...], w_ref[...], (((1,), (1,)), ((), ())),
                        preferred_element_type=jnp.float32)
    z_ref[...] = z
    nrm = jnp.sqrt(jnp.sum(z * z, axis=1, keepdims=True))
    scale = jnp.sqrt(beta_ref[0, 0]) / jnp.maximum(nrm, 1e-6)
    zh_ref[...] = (z * scale).astype(jnp.bfloat16)


def _prep(h, W, beta):
    n = h.shape[0]
    rb = 1000
    grid = n // rb
    return pl.pallas_call(
        _prep_body,
        grid=(grid,),
        in_specs=[
            pl.BlockSpec((1, 1), lambda i: (0, 0)),
            pl.BlockSpec((rb, D), lambda i: (i, 0)),
            pl.BlockSpec((D, D), lambda i: (0, 0)),
        ],
        out_specs=[
            pl.BlockSpec((rb, D), lambda i: (i, 0)),
            pl.BlockSpec((rb, D), lambda i: (i, 0)),
        ],
        out_shape=[
            jax.ShapeDtypeStruct((n, D), jnp.float32),
            jax.ShapeDtypeStruct((n, D), jnp.bfloat16),
        ],
    )(beta.reshape(1, 1), h, W)


def _gram_body(a_ref, b_ref, o_ref):
    o_ref[...] = lax.dot_general(a_ref[...], b_ref[...],
                                 (((1,), (1,)), ((), ())),
                                 preferred_element_type=jnp.float32)


JB = 16


def _gram_body3(a_ref, b_ref, o_ref):
    for k in range(JB):
        o_ref[k, ...] = lax.dot_general(
            a_ref[...], b_ref[pl.ds(k * D, D), :],
            (((1,), (1,)), ((), ())),
            preferred_element_type=jnp.float32)


def _gram(zh):
    n = zh.shape[0]
    bm = 1000
    ncb = pl.cdiv(n, JB * D)
    return pl.pallas_call(
        _gram_body3,
        grid=(n // bm, ncb),
        in_specs=[
            pl.BlockSpec((bm, D), lambda i, j: (i, 0)),
            pl.BlockSpec((JB * D, D), lambda i, j: (j, 0)),
        ],
        out_specs=pl.BlockSpec((JB, bm, D), lambda i, j: (j, i, 0)),
        out_shape=jax.ShapeDtypeStruct((ncb * JB, n, D), jnp.float32),
    )(zh, zh)


def _make_k1(e, n, npad):
    nblk = e // BLK
    iters = pl.cdiv(nblk, NW)
    sl = npad // NSUB
    mesh = plsc.VectorSubcoreMesh(core_axis_name="c", subcore_axis_name="s", num_cores=2, num_subcores=16)

    @functools.partial(
        pl.kernel,
        mesh=mesh,
        compiler_params=pltpu.CompilerParams(needs_layout_passes=False),
        out_type=(
            jax.ShapeDtypeStruct((e,), jnp.float32),
            jax.ShapeDtypeStruct((npad,), jnp.float32),
            jax.ShapeDtypeStruct((npad,), jnp.float32),
        ),
        scratch_types=[
            pltpu.VMEM((BLK,), jnp.int32),
            pltpu.VMEM((BLK,), jnp.int32),
            pltpu.VMEM((BLK,), jnp.int32),
            pltpu.VMEM((BLK,), jnp.int32),
            pltpu.VMEM((BLK,), jnp.int32),
            pltpu.VMEM((BLK,), jnp.int32),
            pltpu.VMEM((BLK, D), jnp.float32),
            pltpu.VMEM((BLK, D), jnp.float32),
            pltpu.VMEM((BLK,), jnp.float32),
            pltpu.VMEM((sl,), jnp.float32),
            pltpu.VMEM_SHARED((npad,), jnp.float32),
            pltpu.SemaphoreType.DMA,
            pltpu.SemaphoreType.DMA,
        ],
    )
    def k1(g2_hbm, src_hbm, dst_hbm, w_hbm, s0_hbm, s1_hbm,
           src_a, src_b, dst_a, dst_b, row_a, row_b, gst_a, gst_b,
           w_v, sbuf_v, s_sh, sem_a, sem_b):
        cid = lax.axis_index("c")
        sid = lax.axis_index("s")
        wid = cid * NSUB + sid
        srcs = [src_a, src_b]
        dsts = [dst_a, dst_b]
        rows_ = [row_a, row_b]
        gsts = [gst_a, gst_b]
        sems = [sem_a, sem_b]

        zero16 = jnp.zeros((16,), jnp.float32)
        def zbody(i, c):
            sbuf_v[pl.ds(i * 16, 16)] = zero16
            return c
        lax.fori_loop(0, sl // 16, zbody, 0)
        pltpu.sync_copy(sbuf_v, s_sh.at[pl.ds(sid * sl, sl)])
        plsc.subcore_barrier()

        nconst = jnp.full((16,), n, jnp.int32)
        seven = jnp.full((16,), 7, jnp.int32)
        mask7 = jnp.full((16,), D - 1, jnp.int32)

        def issue(bq, q):
            baseq = bq * BLK
            pltpu.sync_copy(src_hbm.at[pl.ds(baseq, BLK)], srcs[q])
            pltpu.sync_copy(dst_hbm.at[pl.ds(baseq, BLK)], dsts[q])
            for g in range(BLK // 16):
                ix = pl.ds(g * 16, 16)
                dv = dsts[q][ix]
                rows_[q][ix] = lax.shift_right_logical(dv, seven) * nconst \
                    + srcs[q][ix]
            pltpu.async_copy(g2_hbm.at[rows_[q]], gsts[q], sems[q])

        issue(wid, 0)

        def body2(i2, c):
            for sub in range(2):
                p = sub
                q = 1 - sub
                b = wid + (i2 * 2 + sub) * NW
                bn = b + NW

                @pl.when(bn < nblk)
                def _():
                    issue(bn, q)

                @pl.when(b < nblk)
                def _():
                    base = b * BLK
                    pltpu.make_async_copy(
                        g2_hbm.at[rows_[p]], gsts[p], sems[p]).wait()
                    for g in range(BLK // 16):
                        ix = pl.ds(g * 16, 16)
                        rr = lax.iota(jnp.int32, 16) + g * 16
                        cm = lax.bitwise_and(dsts[p][ix], mask7)
                        val = plsc.load_gather(gsts[p], [rr, cm])
                        w_v[ix] = jnp.exp(val)
                    pltpu.sync_copy(w_v, w_hbm.at[pl.ds(base, BLK)])
                    pltpu.sync_copy(w_v, s_sh.at[dsts[p]], add=True)
            return c

        lax.fori_loop(0, (iters + 1) // 2, body2, 0)
        plsc.subcore_barrier()

        pltpu.sync_copy(s_sh.at[pl.ds(sid * sl, sl)], sbuf_v)

        @pl.when(cid == 0)
        def _():
            pltpu.sync_copy(sbuf_v, s0_hbm.at[pl.ds(sid * sl, sl)])

        @pl.when(cid == 1)
        def _():
            pltpu.sync_copy(sbuf_v, s1_hbm.at[pl.ds(sid * sl, sl)])

    return k1


def _sum_s_body(a_ref, b_ref, o_ref):
    o_ref[...] = a_ref[...] + b_ref[...]


def _sum_s(s0, s1):
    npad = s0.shape[0]
    return pl.pallas_call(
        _sum_s_body,
        out_shape=jax.ShapeDtypeStruct((npad,), jnp.float32),
    )(s0, s1)


def _make_k2(e, npad):
    nblk = e // BLK
    iters = pl.cdiv(nblk, NW)
    sl = npad // NSUB
    rows_per_copy = 128
    mesh = plsc.VectorSubcoreMesh(core_axis_name="c", subcore_axis_name="s", num_cores=2, num_subcores=16)

    @functools.partial(
        pl.kernel,
        mesh=mesh,
        compiler_params=pltpu.CompilerParams(needs_layout_passes=False),
        out_type=(
            jax.ShapeDtypeStruct((npad, D), jnp.float32),
            jax.ShapeDtypeStruct((npad, D), jnp.float32),
        ),
        scratch_types=[
            pltpu.VMEM((BLK,), jnp.int32),
            pltpu.VMEM((BLK,), jnp.int32),
            pltpu.VMEM((BLK,), jnp.int32),
            pltpu.VMEM((BLK,), jnp.int32),
            pltpu.VMEM((BLK,), jnp.float32),
            pltpu.VMEM((BLK,), jnp.float32),
            pltpu.VMEM((BLK,), jnp.float32),
            pltpu.VMEM((BLK,), jnp.float32),
            pltpu.VMEM((BLK, D), jnp.float32),
            pltpu.VMEM((BLK, D), jnp.float32),
            pltpu.VMEM((BLK,), jnp.float32),
            pltpu.VMEM_SHARED((npad, D), jnp.float32),
            pltpu.SemaphoreType.DMA,
            pltpu.SemaphoreType.DMA,
            pltpu.SemaphoreType.DMA,
            pltpu.SemaphoreType.DMA,
        ],
    )
    def k2(z_hbm, src_hbm, dst_hbm, w_hbm, s_hbm,
           out0_hbm, out1_hbm,
           src_a, src_b, dst_a, dst_b, w_a, w_b, sg_a, sg_b, zr_a, zr_b,
           al_v, out_sh, sem_a, sem_b, semg_a, semg_b):
        cid = lax.axis_index("c")
        sid = lax.axis_index("s")
        wid = cid * NSUB + sid
        srcs = [src_a, src_b]
        dsts = [dst_a, dst_b]
        ws = [w_a, w_b]
        sgs = [sg_a, sg_b]
        zrs = [zr_a, zr_b]
        sems = [sem_a, sem_b]
        semgs = [semg_a, semg_b]

        zero16 = jnp.zeros((16,), jnp.float32)
        def zbody(i, c):
            r = i // 8
            cchunk = i % 8
            zr_a[r, pl.ds(cchunk * 16, 16)] = zero16
            return c
        lax.fori_loop(0, rows_per_copy * 8, zbody, 0)
        for j in range(sl // rows_per_copy):
            pltpu.sync_copy(
                zr_a, out_sh.at[pl.ds(sid * sl + j * rows_per_copy,
                                      rows_per_copy)])
        plsc.subcore_barrier()

        def issue(bq, q):
            baseq = bq * BLK
            pltpu.sync_copy(src_hbm.at[pl.ds(baseq, BLK)], srcs[q])
            pltpu.sync_copy(dst_hbm.at[pl.ds(baseq, BLK)], dsts[q])
            pltpu.sync_copy(w_hbm.at[pl.ds(baseq, BLK)], ws[q])
            pltpu.async_copy(s_hbm.at[dsts[q]], sgs[q], semgs[q])
            pltpu.async_copy(z_hbm.at[srcs[q]], zrs[q], sems[q])

        issue(wid, 0)

        def body2(i2, c):
            for sub in range(2):
                p = sub
                q = 1 - sub
                b = wid + (i2 * 2 + sub) * NW
                bn = b + NW

                @pl.when(bn < nblk)
                def _():
                    issue(bn, q)

                @pl.when(b < nblk)
                def _():
                    pltpu.make_async_copy(
                        s_hbm.at[dsts[p]], sgs[p], semgs[p]).wait()
                    pltpu.make_async_copy(
                        z_hbm.at[srcs[p]], zrs[p], sems[p]).wait()
                    for g in range(BLK // 16):
                        ix = pl.ds(g * 16, 16)
                        al_v[ix] = ws[p][ix] / sgs[p][ix]

                    def ebody(t, c2):
                        for u in range(2):
                            ei = t * 2 + u
                            ab = plsc.load_gather(
                                al_v, [jnp.full((16,), ei, jnp.int32)])
                            for cchunk in range(8):
                                ix = pl.ds(cchunk * 16, 16)
                                zrs[p][ei, ix] = zrs[p][ei, ix] * ab
                        return c2

                    lax.fori_loop(0, BLK // 2, ebody, 0)
                    pltpu.sync_copy(zrs[p], out_sh.at[dsts[p]], add=True)
            return c

        lax.fori_loop(0, (iters + 1) // 2, body2, 0)
        plsc.subcore_barrier()

        for j in range(sl // rows_per_copy):
            r0 = sid * sl + j * rows_per_copy
            pltpu.sync_copy(out_sh.at[pl.ds(r0, rows_per_copy)], zr_a)

            @pl.when(cid == 0)
            def _():
                pltpu.sync_copy(zr_a, out0_hbm.at[pl.ds(r0, rows_per_copy)])

            @pl.when(cid == 1)
            def _():
                pltpu.sync_copy(zr_a, out1_hbm.at[pl.ds(r0, rows_per_copy)])

    return k2


def _add_body(a_ref, b_ref, o_ref):
    o_ref[...] = a_ref[...] + b_ref[...]


def _final_add(a, b, n):
    rb = 80
    grid = n // rb
    return pl.pallas_call(
        _add_body,
        grid=(grid,),
        in_specs=[
            pl.BlockSpec((rb, D), lambda i: (i, 0)),
            pl.BlockSpec((rb, D), lambda i: (i, 0)),
        ],
        out_specs=pl.BlockSpec((rb, D), lambda i: (i, 0)),
        out_shape=jax.ShapeDtypeStruct((n, D), jnp.float32),
    )(a, b)


def kernel(h, edge_index, W, beta):
    n = h.shape[0]
    e = edge_index.shape[1]
    npad = ((n + 2047) // 2048) * 2048

    z, zh = _prep(h, W, beta)
    g2 = _gram(zh)
    g2 = g2.reshape(g2.shape[0] * n, D)
    src = edge_index[0]
    dst = edge_index[1]
    w, s0, s1 = _make_k1(e, n, npad)(g2, src, dst)
    s = _sum_s(s0, s1)
    out0, out1 = _make_k2(e, npad)(z, src, dst, w, s)
    return _final_add(out0, out1, n)


# speedup vs baseline: 2.1811x; 1.1259x over previous
import functools

import jax
import jax.numpy as jnp
from jax import lax
from jax.experimental import pallas as pl
from jax.experimental.pallas import tpu as pltpu
from jax.experimental.pallas import tpu_sc as plsc

D = 128
BLK = 128
NW = 32
NSUB = 16


def _prep_body(beta_ref, h_ref, w_ref, z_ref, zh_ref):
    z = lax.dot_general(h_ref[...], w_ref[...], (((1,), (1,)), ((), ())),
                        preferred_element_type=jnp.float32)
    z_ref[...] = z
    nrm = jnp.sqrt(jnp.sum(z * z, axis=1, keepdims=True))
    scale = jnp.sqrt(beta_ref[0, 0]) / jnp.maximum(nrm, 1e-6)
    zh_ref[...] = (z * scale).astype(jnp.bfloat16)


def _prep(h, W, beta):
    n = h.shape[0]
    rb = 1000
    grid = n // rb
    return pl.pallas_call(
        _prep_body,
        grid=(grid,),
        in_specs=[
            pl.BlockSpec((1, 1), lambda i: (0, 0)),
            pl.BlockSpec((rb, D), lambda i: (i, 0)),
            pl.BlockSpec((D, D), lambda i: (0, 0)),
        ],
        out_specs=[
            pl.BlockSpec((rb, D), lambda i: (i, 0)),
            pl.BlockSpec((rb, D), lambda i: (i, 0)),
        ],
        out_shape=[
            jax.ShapeDtypeStruct((n, D), jnp.float32),
            jax.ShapeDtypeStruct((n, D), jnp.bfloat16),
        ],
    )(beta.reshape(1, 1), h, W)


def _gram_body(a_ref, b_ref, o_ref):
    o_ref[...] = lax.dot_general(a_ref[...], b_ref[...],
                                 (((1,), (1,)), ((), ())),
                                 preferred_element_type=jnp.float32)


JB = 16


def _gram_body3(a_ref, b_ref, o_ref):
    for k in range(JB):
        o_ref[k, ...] = lax.dot_general(
            a_ref[...], b_ref[pl.ds(k * D, D), :],
            (((1,), (1,)), ((), ())),
            preferred_element_type=jnp.float32)


def _gram(zh):
    n = zh.shape[0]
    bm = 1000
    ncb = pl.cdiv(n, JB * D)
    return pl.pallas_call(
        _gram_body3,
        grid=(n // bm, ncb),
        in_specs=[
            pl.BlockSpec((bm, D), lambda i, j: (i, 0)),
            pl.BlockSpec((JB * D, D), lambda i, j: (j, 0)),
        ],
        out_specs=pl.BlockSpec((JB, bm, D), lambda i, j: (j, i, 0)),
        out_shape=jax.ShapeDtypeStruct((ncb * JB, n, D), jnp.float32),
    )(zh, zh)


def _make_k1(e, n, npad):
    nblk = e // BLK
    iters = pl.cdiv(nblk, NW)
    sl = npad // NSUB
    mesh = plsc.VectorSubcoreMesh(core_axis_name="c", subcore_axis_name="s", num_cores=2, num_subcores=16)

    @functools.partial(
        pl.kernel,
        mesh=mesh,
        compiler_params=pltpu.CompilerParams(needs_layout_passes=False),
        out_type=(
            jax.ShapeDtypeStruct((e,), jnp.float32),
            jax.ShapeDtypeStruct((npad,), jnp.float32),
            jax.ShapeDtypeStruct((npad,), jnp.float32),
        ),
        scratch_types=[
            pltpu.VMEM((BLK,), jnp.int32),
            pltpu.VMEM((BLK,), jnp.int32),
            pltpu.VMEM((BLK,), jnp.int32),
            pltpu.VMEM((BLK,), jnp.int32),
            pltpu.VMEM((BLK,), jnp.int32),
            pltpu.VMEM((BLK,), jnp.int32),
            pltpu.VMEM((BLK, D), jnp.float32),
            pltpu.VMEM((BLK, D), jnp.float32),
            pltpu.VMEM((BLK,), jnp.float32),
            pltpu.VMEM((sl,), jnp.float32),
            pltpu.VMEM_SHARED((npad,), jnp.float32),
            pltpu.SemaphoreType.DMA,
            pltpu.SemaphoreType.DMA,
        ],
    )
    def k1(g2_hbm, ei_hbm, w_hbm, s0_hbm, s1_hbm,
           src_a, src_b, dst_a, dst_b, row_a, row_b, gst_a, gst_b,
           w_v, sbuf_v, s_sh, sem_a, sem_b):
        cid = lax.axis_index("c")
        sid = lax.axis_index("s")
        wid = cid * NSUB + sid
        srcs = [src_a, src_b]
        dsts = [dst_a, dst_b]
        rows_ = [row_a, row_b]
        gsts = [gst_a, gst_b]
        sems = [sem_a, sem_b]

        zero16 = jnp.zeros((16,), jnp.float32)
        def zbody(i, c):
            sbuf_v[pl.ds(i * 16, 16)] = zero16
            return c
        lax.fori_loop(0, sl // 16, zbody, 0)
        pltpu.sync_copy(sbuf_v, s_sh.at[pl.ds(sid * sl, sl)])
        plsc.subcore_barrier()

        nconst = jnp.full((16,), n, jnp.int32)
        seven = jnp.full((16,), 7, jnp.int32)
        mask7 = jnp.full((16,), D - 1, jnp.int32)

        def issue(bq, q):
            baseq = bq * BLK
            pltpu.sync_copy(ei_hbm.at[0, pl.ds(baseq, BLK)], srcs[q])
            pltpu.sync_copy(ei_hbm.at[1, pl.ds(baseq, BLK)], dsts[q])
            for g in range(BLK // 16):
                ix = pl.ds(g * 16, 16)
                dv = dsts[q][ix]
                rows_[q][ix] = lax.shift_right_logical(dv, seven) * nconst \
                    + srcs[q][ix]
            pltpu.async_copy(g2_hbm.at[rows_[q]], gsts[q], sems[q])

        issue(wid, 0)

        def body2(i2, c):
            for sub in range(2):
                p = sub
                q = 1 - sub
                b = wid + (i2 * 2 + sub) * NW
                bn = b + NW

                @pl.when(bn < nblk)
                def _():
                    issue(bn, q)

                @pl.when(b < nblk)
                def _():
                    base = b * BLK
                    pltpu.make_async_copy(
                        g2_hbm.at[rows_[p]], gsts[p], sems[p]).wait()
                    for g in range(BLK // 16):
                        ix = pl.ds(g * 16, 16)
                        rr = lax.iota(jnp.int32, 16) + g * 16
                        cm = lax.bitwise_and(dsts[p][ix], mask7)
                        val = plsc.load_gather(gsts[p], [rr, cm])
                        w_v[ix] = jnp.exp(val)
                    pltpu.sync_copy(w_v, w_hbm.at[pl.ds(base, BLK)])
                    pltpu.sync_copy(w_v, s_sh.at[dsts[p]], add=True)
            return c

        lax.fori_loop(0, (iters + 1) // 2, body2, 0)
        plsc.subcore_barrier()

        pltpu.sync_copy(s_sh.at[pl.ds(sid * sl, sl)], sbuf_v)

        @pl.when(cid == 0)
        def _():
            pltpu.sync_copy(sbuf_v, s0_hbm.at[pl.ds(sid * sl, sl)])

        @pl.when(cid == 1)
        def _():
            pltpu.sync_copy(sbuf_v, s1_hbm.at[pl.ds(sid * sl, sl)])

    return k1


def _sum_s_body(a_ref, b_ref, o_ref):
    o_ref[...] = a_ref[...] + b_ref[...]


def _sum_s(s0, s1):
    npad = s0.shape[0]
    return pl.pallas_call(
        _sum_s_body,
        out_shape=jax.ShapeDtypeStruct((npad,), jnp.float32),
    )(s0, s1)


def _make_k2(e, npad):
    nblk = e // BLK
    iters = pl.cdiv(nblk, NW)
    sl = npad // NSUB
    rows_per_copy = 128
    mesh = plsc.VectorSubcoreMesh(core_axis_name="c", subcore_axis_name="s", num_cores=2, num_subcores=16)

    @functools.partial(
        pl.kernel,
        mesh=mesh,
        compiler_params=pltpu.CompilerParams(needs_layout_passes=False),
        out_type=(
            jax.ShapeDtypeStruct((npad, D), jnp.float32),
            jax.ShapeDtypeStruct((npad, D), jnp.float32),
        ),
        scratch_types=[
            pltpu.VMEM((BLK,), jnp.int32),
            pltpu.VMEM((BLK,), jnp.int32),
            pltpu.VMEM((BLK,), jnp.int32),
            pltpu.VMEM((BLK,), jnp.int32),
            pltpu.VMEM((BLK,), jnp.float32),
            pltpu.VMEM((BLK,), jnp.float32),
            pltpu.VMEM((BLK,), jnp.float32),
            pltpu.VMEM((BLK,), jnp.float32),
            pltpu.VMEM((BLK, D), jnp.float32),
            pltpu.VMEM((BLK, D), jnp.float32),
            pltpu.VMEM((BLK,), jnp.float32),
            pltpu.VMEM_SHARED((npad, D), jnp.float32),
            pltpu.SemaphoreType.DMA,
            pltpu.SemaphoreType.DMA,
            pltpu.SemaphoreType.DMA,
            pltpu.SemaphoreType.DMA,
            pltpu.SemaphoreType.DMA,
            pltpu.SemaphoreType.DMA,
        ],
    )
    def k2(z_hbm, ei_hbm, w_hbm, s_hbm,
           out0_hbm, out1_hbm,
           src_a, src_b, dst_a, dst_b, w_a, w_b, sg_a, sg_b, zr_a, zr_b,
           al_v, out_sh, sem_a, sem_b, semg_a, semg_b, semsc_a, semsc_b):
        cid = lax.axis_index("c")
        sid = lax.axis_index("s")
        wid = cid * NSUB + sid
        srcs = [src_a, src_b]
        dsts = [dst_a, dst_b]
        ws = [w_a, w_b]
        sgs = [sg_a, sg_b]
        zrs = [zr_a, zr_b]
        sems = [sem_a, sem_b]
        semgs = [semg_a, semg_b]
        semscs = [semsc_a, semsc_b]

        zero16 = jnp.zeros((16,), jnp.float32)
        def zbody(i, c):
            r = i // 8
            cchunk = i % 8
            zr_a[r, pl.ds(cchunk * 16, 16)] = zero16
            return c
        lax.fori_loop(0, rows_per_copy * 8, zbody, 0)
        for j in range(sl // rows_per_copy):
            pltpu.sync_copy(
                zr_a, out_sh.at[pl.ds(sid * sl + j * rows_per_copy,
                                      rows_per_copy)])
        plsc.subcore_barrier()

        def issue(bq, q):
            baseq = bq * BLK
            pltpu.sync_copy(ei_hbm.at[0, pl.ds(baseq, BLK)], srcs[q])
            pltpu.sync_copy(ei_hbm.at[1, pl.ds(baseq, BLK)], dsts[q])
            pltpu.sync_copy(w_hbm.at[pl.ds(baseq, BLK)], ws[q])
            pltpu.async_copy(s_hbm.at[dsts[q]], sgs[q], semgs[q])
            pltpu.async_copy(z_hbm.at[srcs[q]], zrs[q], sems[q])

        def drain_scatter(q):
            pltpu.make_async_copy(zrs[q], out_sh.at[dsts[q]], semscs[q]).wait()

        def compute(p):
            pltpu.make_async_copy(s_hbm.at[dsts[p]], sgs[p], semgs[p]).wait()
            pltpu.make_async_copy(z_hbm.at[srcs[p]], zrs[p], sems[p]).wait()
            for g in range(BLK // 16):
                ix = pl.ds(g * 16, 16)
                al_v[ix] = ws[p][ix] / sgs[p][ix]

            def ebody(t, c2):
                for u in range(2):
                    ei = t * 2 + u
                    ab = plsc.load_gather(
                        al_v, [jnp.full((16,), ei, jnp.int32)])
                    for cchunk in range(8):
                        ix = pl.ds(cchunk * 16, 16)
                        zrs[p][ei, ix] = zrs[p][ei, ix] * ab
                return c2

            lax.fori_loop(0, BLK // 2, ebody, 0)
            pltpu.async_copy(zrs[p], out_sh.at[dsts[p]], semscs[p], add=True)

        issue(wid, 0)
        b1 = wid + NW

        @pl.when(b1 < nblk)
        def _():
            issue(b1, 1)
        compute(0)

        @pl.when(b1 < nblk)
        def _():
            b2 = b1 + NW

            @pl.when(b2 < nblk)
            def _():
                drain_scatter(0)
                issue(b2, 0)
            compute(1)

        def body2(i2, c):
            for sub in range(2):
                p = sub
                q = 1 - sub
                b = wid + (2 + i2 * 2 + sub) * NW
                bn = b + NW

                @pl.when(b < nblk)
                def _():
                    @pl.when(bn < nblk)
                    def _():
                        drain_scatter(q)
                        issue(bn, q)
                    compute(p)
            return c

        lax.fori_loop(0, (iters - 1) // 2, body2, 0)
        for q in range(2):
            drain_scatter(q)
        plsc.subcore_barrier()

        for j in range(sl // rows_per_copy):
            r0 = sid * sl + j * rows_per_copy
            pltpu.sync_copy(out_sh.at[pl.ds(r0, rows_per_copy)], zr_a)

            @pl.when(cid == 0)
            def _():
                pltpu.sync_copy(zr_a, out0_hbm.at[pl.ds(r0, rows_per_copy)])

            @pl.when(cid == 1)
            def _():
                pltpu.sync_copy(zr_a, out1_hbm.at[pl.ds(r0, rows_per_copy)])

    return k2


def _add_body(a_ref, b_ref, o_ref):
    o_ref[...] = a_ref[...] + b_ref[...]


def _final_add(a, b, n):
    rb = 2000
    grid = n // rb
    return pl.pallas_call(
        _add_body,
        grid=(grid,),
        in_specs=[
            pl.BlockSpec((rb, D), lambda i: (i, 0)),
            pl.BlockSpec((rb, D), lambda i: (i, 0)),
        ],
        out_specs=pl.BlockSpec((rb, D), lambda i: (i, 0)),
        out_shape=jax.ShapeDtypeStruct((n, D), jnp.float32),
    )(a, b)


def kernel(h, edge_index, W, beta):
    n = h.shape[0]
    e = edge_index.shape[1]
    npad = ((n + 2047) // 2048) * 2048

    z, zh = _prep(h, W, beta)
    g2 = _gram(zh)
    g2 = g2.reshape(g2.shape[0] * n, D)
    w, s0, s1 = _make_k1(e, n, npad)(g2, edge_index)
    s = _sum_s(s0, s1)
    out0, out1 = _make_k2(e, npad)(z, edge_index, w, s)
    return _final_add(out0, out1, n)
